# Initial kernel scaffold; baseline (speedup 1.0000x reference)
import functools

import jax
import jax.numpy as jnp
from jax import lax
from jax.experimental import pallas as pl
from jax.experimental.pallas import tpu as pltpu
from jax.experimental.pallas import tpu_sc as plsc

N = 10000
E = 320000
D = 128
NUM_RELS = 64

NC = 2
NS = 16
NW = NC * NS
E_PER_W = E // NW
G = 80
NG = E_PER_W // G
DACC = 144
ROWS_PER_TILE = N // NS
RCHUNK = 125


def _tc_pre_body(h_ref, w_ref, aw_ref, z_ref, st_ref):
    z = jnp.dot(h_ref[...], w_ref[...], preferred_element_type=jnp.float32)
    z_ref[...] = z
    a = jnp.concatenate([aw_ref[:D, :], aw_ref[D:, :]], axis=1)
    st_ref[...] = jnp.dot(z, a, preferred_element_type=jnp.float32)


_tc_pre = pl.pallas_call(
    _tc_pre_body,
    out_shape=(
        jax.ShapeDtypeStruct((N, D), jnp.float32),
        jax.ShapeDtypeStruct((N, 2), jnp.float32),
    ),
)


def _sc_edges_body(z_hbm, s_hbm, t_hbm, rel_hbm, src_hbm, dst_hbm, et_hbm,
                   out_hbm,
                   s_tab, t_tab, rel_tab, src_buf, dst_buf, et_buf,
                   zbuf, sbuf, wbuf, exbuf, bounce, acc):
    cid = lax.axis_index("c")
    sid = lax.axis_index("s")
    wid = cid * NS + sid

    pltpu.sync_copy(s_hbm, s_tab)
    pltpu.sync_copy(t_hbm, t_tab)
    pltpu.sync_copy(rel_hbm, rel_tab)
    pltpu.sync_copy(src_hbm.at[wid], src_buf)
    pltpu.sync_copy(dst_hbm.at[wid], dst_buf)
    pltpu.sync_copy(et_hbm.at[wid], et_buf)

    zeros16 = jnp.zeros((16,), jnp.float32)

    def _zero_row(i, carry):
        for j in range(DACC // 16):
            bounce[i, pl.ds(j * 16, 16)] = zeros16
        return carry
    lax.fori_loop(0, RCHUNK, _zero_row, 0)
    for k in range(ROWS_PER_TILE // RCHUNK):
        pltpu.sync_copy(bounce, acc.at[pl.ds(sid * ROWS_PER_TILE + k * RCHUNK, RCHUNK)])

    def _zero_pad(i, carry):
        sbuf[i, pl.ds(D, 16)] = zeros16
        return carry
    lax.fori_loop(0, G, _zero_pad, 0)

    plsc.subcore_barrier()

    def _group(g, carry):
        base = g * G
        pltpu.sync_copy(z_hbm.at[src_buf.at[pl.ds(base, G)]], zbuf)
        for v in range(G // 16):
            off = base + v * 16
            src16 = src_buf[pl.ds(off, 16)]
            dst16 = dst_buf[g, pl.ds(v * 16, 16)]
            et16 = et_buf[pl.ds(off, 16)]
            s_src = plsc.load_gather(s_tab, [src16])
            t_dst = plsc.load_gather(t_tab, [dst16])
            rel16 = plsc.load_gather(rel_tab, [et16])
            a = s_src + t_dst
            e = jnp.where(a > 0.0, a, a * 0.01)
            ex = jnp.exp(e)
            wbuf[pl.ds(v * 16, 16)] = rel16 * ex
            exbuf[pl.ds(v * 16, 16)] = ex
        for j in range(G):
            wv = jnp.full((16,), wbuf[j])
            for k in range(D // 16):
                sbuf[j, pl.ds(k * 16, 16)] = zbuf[j, pl.ds(k * 16, 16)] * wv
            sbuf[j, D] = exbuf[j]
        pltpu.sync_copy(sbuf, acc.at[dst_buf.at[g]], add=True)
        return carry

    lax.fori_loop(0, NG, _group, 0)

    plsc.subcore_barrier()

    for k in range(ROWS_PER_TILE // RCHUNK):
        r0 = sid * ROWS_PER_TILE + k * RCHUNK
        pltpu.sync_copy(acc.at[pl.ds(r0, RCHUNK)], bounce)
        pltpu.sync_copy(bounce, out_hbm.at[cid].at[pl.ds(r0, RCHUNK)])


_sc_edges = functools.partial(
    pl.kernel,
    out_type=jax.ShapeDtypeStruct((NC, N, DACC), jnp.float32),
    mesh=plsc.VectorSubcoreMesh(core_axis_name="c", subcore_axis_name="s"),
    scratch_types=[
        pltpu.VMEM((N,), jnp.float32),
        pltpu.VMEM((N,), jnp.float32),
        pltpu.VMEM((NUM_RELS,), jnp.float32),
        pltpu.VMEM((E_PER_W,), jnp.int32),
        pltpu.VMEM((NG, G), jnp.int32),
        pltpu.VMEM((E_PER_W,), jnp.int32),
        pltpu.VMEM((G, D), jnp.float32),
        pltpu.VMEM((G, DACC), jnp.float32),
        pltpu.VMEM((G,), jnp.float32),
        pltpu.VMEM((G,), jnp.float32),
        pltpu.VMEM((RCHUNK, DACC), jnp.float32),
        pltpu.VMEM_SHARED((N, DACC), jnp.float32),
    ],
)(_sc_edges_body)


def _tc_post_body(p_ref, o_ref):
    a = p_ref[0] + p_ref[1]
    w = a[:, :D]
    den = a[:, D:D + 1]
    o_ref[...] = jnp.where(den > 0.0, w / den, 0.0)


_tc_post = pl.pallas_call(
    _tc_post_body,
    out_shape=jax.ShapeDtypeStruct((N, D), jnp.float32),
)


def kernel(h, edge_index, edge_type, W, rel_emb, attn_w):
    z, st = _tc_pre(h, W, attn_w)
    s = st[:, 0]
    t = st[:, 1]
    src = edge_index[0].reshape(NW, E_PER_W)
    dst = edge_index[1].reshape(NW, NG, G)
    et = edge_type.reshape(NW, E_PER_W)
    rel = rel_emb[:, 0]
    p = _sc_edges(z, s, t, rel, src, dst, et)
    return _tc_post(p)


# speedup vs baseline: 18.2550x; 18.2550x over previous
import functools

import jax
import jax.numpy as jnp
from jax import lax
from jax.experimental import pallas as pl
from jax.experimental.pallas import tpu as pltpu
from jax.experimental.pallas import tpu_sc as plsc

N = 10000
E = 320000
D = 128
NUM_RELS = 64

NC = 2
NS = 16
NW = NC * NS
E_PER_W = E // NW
NVEC = E_PER_W // 16
G = 80
RANGE = 5120
NPASS = 2
OUTROWS = NPASS * RANGE
ACC_PER_TILE = RANGE // NS
RB = 64
EIDX_CAP = 10080


def _tc_pre_body(h_ref, w_ref, aw_ref, z_ref, st_ref):
    z = jnp.dot(h_ref[...], w_ref[...], preferred_element_type=jnp.float32)
    z_ref[...] = z
    a = jnp.concatenate([aw_ref[:D, :], aw_ref[D:, :]], axis=1)
    st_ref[...] = jnp.dot(z, a, preferred_element_type=jnp.float32)


_tc_pre = pl.pallas_call(
    _tc_pre_body,
    out_shape=(
        jax.ShapeDtypeStruct((N, D), jnp.float32),
        jax.ShapeDtypeStruct((N, 2), jnp.float32),
    ),
)


def _sc_edges_body(z_hbm, s_hbm, t_hbm, rel_hbm, pk_hbm, et_hbm,
                   out_hbm, outden_hbm,
                   s_tab, t_tab, rel_tab, pk_buf, eidx_buf, w_buf,
                   sidx, didx, wstage, den_tab, zbuf, zerosrc,
                   bounce, acc):
    cid = lax.axis_index("c")
    sid = lax.axis_index("s")
    wid = cid * NS + sid

    pltpu.sync_copy(s_hbm, s_tab)
    pltpu.sync_copy(t_hbm, t_tab)
    pltpu.sync_copy(rel_hbm, rel_tab)
    pltpu.sync_copy(pk_hbm.at[wid], pk_buf)
    pltpu.sync_copy(et_hbm.at[wid], eidx_buf)

    zeros16 = jnp.zeros((16,), jnp.float32)
    lane = lax.iota(jnp.int32, 16)

    def _zero_den(i, carry):
        den_tab[pl.ds(i * 16, 16)] = zeros16
        return carry
    lax.fori_loop(0, N // 16, _zero_den, 0)

    def _zero_row(i, carry):
        for j in range(D // 16):
            zerosrc[i, pl.ds(j * 16, 16)] = zeros16
        return carry
    lax.fori_loop(0, RB, _zero_row, 0)
    for k in range(ACC_PER_TILE // RB):
        pltpu.sync_copy(zerosrc, acc.at[pl.ds(sid * ACC_PER_TILE + k * RB, RB)])

    def _score(i, carry):
        off = i * 16
        pk16 = pk_buf[pl.ds(off, 16)]
        src16 = pk16 & 0xFFFF
        dst16 = lax.shift_right_logical(pk16, 16)
        et16 = eidx_buf[pl.ds(off, 16)]
        s_src = plsc.load_gather(s_tab, [src16])
        t_dst = plsc.load_gather(t_tab, [dst16])
        rel16 = plsc.load_gather(rel_tab, [et16])
        a = s_src + t_dst
        e = jnp.where(a > 0.0, a, a * 0.01)
        ex = jnp.exp(e)
        plsc.addupdate_scatter(den_tab, [dst16], ex)
        w_buf[pl.ds(off, 16)] = rel16 * ex
        return carry
    lax.fori_loop(0, NVEC, _score, 0)

    plsc.subcore_barrier()

    for p in range(NPASS):
        lo = p * RANGE

        def _compact(i, cnt):
            off = i * 16
            dst16 = lax.shift_right_logical(pk_buf[pl.ds(off, 16)], 16)
            m = (dst16 >= lo) & (dst16 < lo + RANGE)
            plsc.store_compressed(eidx_buf.at[pl.ds(cnt, 16)], off + lane, mask=m)
            return cnt + plsc.all_reduce_population_count(m)[0]
        cnt = lax.fori_loop(0, NVEC, _compact, 0)
        ng = (cnt + (G - 1)) // G

        def _group(g, carry):
            base = g * G
            for v in range(G // 16):
                gbase = base + v * 16
                e16 = eidx_buf[pl.ds(gbase, 16)]
                valid = (gbase + lane) < cnt
                e16 = jnp.where(valid, e16, 0)
                pk16 = plsc.load_gather(pk_buf, [e16])
                src16 = pk16 & 0xFFFF
                dst16 = lax.shift_right_logical(pk16, 16)
                w16 = plsc.load_gather(w_buf, [e16])
                sidx[0, pl.ds(v * 16, 16)] = jnp.where(valid, src16, 0)
                didx[0, pl.ds(v * 16, 16)] = jnp.where(valid, dst16 - lo, 0)
                wstage[pl.ds(v * 16, 16)] = jnp.where(valid, w16, 0.0)
            pltpu.sync_copy(z_hbm.at[sidx.at[0]], zbuf)
            for v in range(G // 16):
                w16 = wstage[pl.ds(v * 16, 16)]
                for jj in range(16):
                    j = v * 16 + jj
                    wv = jnp.full((16,), w16[jj])
                    for k in range(D // 16):
                        zbuf[j, pl.ds(k * 16, 16)] = zbuf[j, pl.ds(k * 16, 16)] * wv
            pltpu.sync_copy(zbuf, acc.at[didx.at[0]], add=True)
            return carry
        lax.fori_loop(0, ng, _group, 0)

        plsc.subcore_barrier()

        for k in range(ACC_PER_TILE // RB):
            r0 = sid * ACC_PER_TILE + k * RB
            pltpu.sync_copy(acc.at[pl.ds(r0, RB)], bounce)
            pltpu.sync_copy(bounce, out_hbm.at[cid].at[pl.ds(p * RANGE + r0, RB)])
            if p < NPASS - 1:
                pltpu.sync_copy(zerosrc, acc.at[pl.ds(r0, RB)])
        if p < NPASS - 1:
            plsc.subcore_barrier()

    pltpu.sync_copy(den_tab, outden_hbm.at[cid].at[sid])


_sc_edges = functools.partial(
    pl.kernel,
    out_type=(
        jax.ShapeDtypeStruct((NC, OUTROWS, D), jnp.float32),
        jax.ShapeDtypeStruct((NC, NS, N), jnp.float32),
    ),
    mesh=plsc.VectorSubcoreMesh(core_axis_name="c", subcore_axis_name="s"),
    compiler_params=pltpu.CompilerParams(needs_layout_passes=False),
    scratch_types=[
        pltpu.VMEM((N,), jnp.float32),
        pltpu.VMEM((N,), jnp.float32),
        pltpu.VMEM((NUM_RELS,), jnp.float32),
        pltpu.VMEM((E_PER_W,), jnp.int32),
        pltpu.VMEM((EIDX_CAP,), jnp.int32),
        pltpu.VMEM((E_PER_W,), jnp.float32),
        pltpu.VMEM((1, G), jnp.int32),
        pltpu.VMEM((1, G), jnp.int32),
        pltpu.VMEM((G,), jnp.float32),
        pltpu.VMEM((N,), jnp.float32),
        pltpu.VMEM((G, D), jnp.float32),
        pltpu.VMEM((RB, D), jnp.float32),
        pltpu.VMEM((RB, D), jnp.float32),
        pltpu.VMEM_SHARED((RANGE, D), jnp.float32),
    ],
)(_sc_edges_body)


def _tc_post_body(p_ref, dn_ref, o_ref):
    ones = jnp.ones((NW, 1), jnp.float32)
    den = lax.dot_general(dn_ref[...], ones, (((0,), (0,)), ((), ())),
                          preferred_element_type=jnp.float32)
    w = p_ref[0, :N] + p_ref[1, :N]
    o_ref[...] = jnp.where(den > 0.0, w / den, 0.0)


_tc_post = pl.pallas_call(
    _tc_post_body,
    out_shape=jax.ShapeDtypeStruct((N, D), jnp.float32),
)


def kernel(h, edge_index, edge_type, W, rel_emb, attn_w):
    z, st = _tc_pre(h, W, attn_w)
    s = st[:, 0]
    t = st[:, 1]
    pk = (edge_index[0] | (edge_index[1] << 16)).reshape(NW, E_PER_W)
    et = jnp.pad(edge_type.reshape(NW, E_PER_W), ((0, 0), (0, EIDX_CAP - E_PER_W)))
    rel = rel_emb[:, 0]
    p, dn = _sc_edges(z, s, t, rel, pk, et)
    return _tc_post(p, dn.reshape(NW, N))
